# Initial kernel scaffold; baseline (speedup 1.0000x reference)
import jax
import jax.numpy as jnp
from jax.experimental import pallas as pl


def kernel(inputs, token_table, pos_embedding):
    raise NotImplementedError("write your pallas kernel here")


# speedup vs baseline: 1.0002x; 1.0002x over previous
import functools

import jax
import jax.numpy as jnp
import numpy as np
from jax import lax
from jax.experimental import pallas as pl
from jax.experimental.pallas import tpu as pltpu
from jax.experimental.pallas import tpu_sc as plsc

VOCAB = 100000
D = 768
B = 4
S = 2048
N_ROWS = B * S

_info = plsc.get_sparse_core_info()
NC, NS, L = _info.num_cores, _info.num_subcores, _info.num_lanes
NW = NC * NS
ROWS_PER_W = N_ROWS // NW
CHUNK = 64
NCHUNKS = ROWS_PER_W // CHUNK
GROUPS = D // L

SCALE = np.float32(np.sqrt(np.float32(D)))

_mesh = plsc.VectorSubcoreMesh(core_axis_name="c", subcore_axis_name="s")


@functools.partial(
    pl.kernel,
    mesh=_mesh,
    out_type=jax.ShapeDtypeStruct((N_ROWS, D), jnp.float32),
    scratch_types=[
        pltpu.VMEM((ROWS_PER_W,), jnp.int32),
        pltpu.VMEM((CHUNK, D), jnp.float32),
        pltpu.VMEM((CHUNK, D), jnp.float32),
        pltpu.SemaphoreType.DMA,
    ],
)
def _embed_kernel(idx_hbm, table_hbm, pos_hbm, out_hbm, idx_v, x_v, pos_v, sem):
    wid = lax.axis_index("s") * NC + lax.axis_index("c")
    base = wid * ROWS_PER_W
    pos_base = (wid % (S // ROWS_PER_W)) * ROWS_PER_W

    pltpu.sync_copy(idx_hbm.at[pl.ds(base, ROWS_PER_W)], idx_v)

    for c in range(NCHUNKS):
        pltpu.sync_copy(pos_hbm.at[pl.ds(pos_base + c * CHUNK, CHUNK), :], pos_v)
        pltpu.async_copy(
            table_hbm.at[idx_v.at[pl.ds(c * CHUNK, CHUNK)]], x_v, sem
        ).wait()

        def row_body(r, _):
            for j in range(GROUPS):
                sl = pl.ds(j * L, L)
                x_v[r, sl] = x_v[r, sl] * SCALE + pos_v[r, sl]
            return 0

        lax.fori_loop(0, CHUNK, row_body, 0)

        pltpu.sync_copy(x_v, out_hbm.at[pl.ds(base + c * CHUNK, CHUNK), :])


def kernel(inputs, token_table, pos_embedding):
    idx = inputs.astype(jnp.int32).reshape(N_ROWS)
    out = _embed_kernel(idx, token_table, pos_embedding)
    return out.reshape(B, S, D)


# speedup vs baseline: 1.2290x; 1.2287x over previous
import functools

import jax
import jax.numpy as jnp
import numpy as np
from jax import lax
from jax.experimental import pallas as pl
from jax.experimental.pallas import tpu as pltpu
from jax.experimental.pallas import tpu_sc as plsc

VOCAB = 100000
D = 768
B = 4
S = 2048
N_ROWS = B * S

_info = plsc.get_sparse_core_info()
NC, NS, L = _info.num_cores, _info.num_subcores, _info.num_lanes
NW = NC * NS
ROWS_PER_W = N_ROWS // NW
CHUNK = 32
NCHUNKS = ROWS_PER_W // CHUNK
GROUPS = D // L

SCALE = np.float32(np.sqrt(np.float32(D)))

_mesh = plsc.VectorSubcoreMesh(core_axis_name="c", subcore_axis_name="s")


@functools.partial(
    pl.kernel,
    mesh=_mesh,
    out_type=jax.ShapeDtypeStruct((N_ROWS, D), jnp.float32),
    scratch_types=[
        pltpu.VMEM((ROWS_PER_W,), jnp.int32),
        pltpu.VMEM((CHUNK, D), jnp.float32),
        pltpu.VMEM((CHUNK, D), jnp.float32),
        pltpu.VMEM((CHUNK, D), jnp.float32),
        pltpu.VMEM((CHUNK, D), jnp.float32),
        pltpu.SemaphoreType.DMA,
        pltpu.SemaphoreType.DMA,
        pltpu.SemaphoreType.DMA,
        pltpu.SemaphoreType.DMA,
        pltpu.SemaphoreType.DMA,
        pltpu.SemaphoreType.DMA,
    ],
)
def _embed_kernel(
    idx_hbm, table_hbm, pos_hbm, out_hbm,
    idx_v, x0, x1, p0, p1, g0, g1, ps0, ps1, o0, o1,
):
    wid = lax.axis_index("s") * NC + lax.axis_index("c")
    base = wid * ROWS_PER_W
    pos_base = (wid % (S // ROWS_PER_W)) * ROWS_PER_W

    pltpu.sync_copy(idx_hbm.at[pl.ds(base, ROWS_PER_W)], idx_v)

    xb = (x0, x1)
    pb = (p0, p1)
    gsem = (g0, g1)
    psem = (ps0, ps1)
    osem = (o0, o1)

    def start_in(c, b):
        gcp = pltpu.async_copy(
            table_hbm.at[idx_v.at[pl.ds(c * CHUNK, CHUNK)]], xb[b], gsem[b]
        )
        pcp = pltpu.async_copy(
            pos_hbm.at[pl.ds(pos_base + c * CHUNK, CHUNK), :], pb[b], psem[b]
        )
        return gcp, pcp

    pending_in = {0: start_in(0, 0)}
    pending_out = {}

    for c in range(NCHUNKS):
        b = c % 2
        if c + 1 < NCHUNKS:
            nb = (c + 1) % 2
            if nb in pending_out:
                pending_out.pop(nb).wait()
            pending_in[c + 1] = start_in(c + 1, nb)

        gcp, pcp = pending_in.pop(c)
        gcp.wait()
        pcp.wait()

        x_v = xb[b]
        pos_v = pb[b]

        def row_body(r, _):
            for j in range(GROUPS):
                sl = pl.ds(j * L, L)
                x_v[r, sl] = x_v[r, sl] * SCALE + pos_v[r, sl]
            return 0

        lax.fori_loop(0, CHUNK, row_body, 0)

        pending_out[b] = pltpu.async_copy(
            x_v, out_hbm.at[pl.ds(base + c * CHUNK, CHUNK), :], osem[b]
        )

    for b in list(pending_out):
        pending_out.pop(b).wait()


def kernel(inputs, token_table, pos_embedding):
    idx = inputs.astype(jnp.int32).reshape(N_ROWS)
    out = _embed_kernel(idx, token_table, pos_embedding)
    return out.reshape(B, S, D)
